# Initial kernel scaffold; baseline (speedup 1.0000x reference)
import functools

import jax
import jax.numpy as jnp
from jax import lax
from jax.experimental import pallas as pl
from jax.experimental.pallas import tpu as pltpu
from jax.experimental.pallas import tpu_sc as plsc

_PAD = 1
_NC = 2
_NS = 16
_NW = _NC * _NS
_CHUNK = 16


def _positions_body(tok_ref, pos_ref):
    t = tok_ref[...]
    mask = (t != _PAD).astype(jnp.int32)
    cs = jnp.cumsum(mask, axis=1)
    pos_ref[...] = cs * mask + _PAD


def _compute_positions(tokens):
    B, S = tokens.shape
    return pl.pallas_call(
        _positions_body,
        out_shape=jax.ShapeDtypeStruct((B, S), jnp.int32),
    )(tokens)


@functools.lru_cache(maxsize=None)
def _make_gather(n_rows, dim):
    b_per_w = n_rows // _NW
    n_chunks = b_per_w // _CHUNK
    mesh = plsc.VectorSubcoreMesh(
        core_axis_name="c", subcore_axis_name="s",
        num_cores=_NC, num_subcores=_NS)

    @functools.partial(
        pl.kernel,
        out_type=jax.ShapeDtypeStruct((n_rows, dim), jnp.float32),
        mesh=mesh,
        scratch_types=[
            pltpu.VMEM((b_per_w,), jnp.int32),
            pltpu.VMEM((_CHUNK, dim), jnp.float32),
            pltpu.VMEM((_CHUNK, dim), jnp.float32),
            pltpu.SemaphoreType.DMA,
            pltpu.SemaphoreType.DMA,
            pltpu.SemaphoreType.DMA,
        ],
    )
    def gather(pos_hbm, table_hbm, out_hbm, idx_v, buf0, buf1, gsem, osem0, osem1):
        wid = lax.axis_index("s") * _NC + lax.axis_index("c")
        base = wid * b_per_w
        pltpu.sync_copy(pos_hbm.at[pl.ds(base, b_per_w)], idx_v)

        bufs = (buf0, buf1)
        osems = (osem0, osem1)

        def chunk(g, b):
            off = g * _CHUNK
            pltpu.async_copy(
                table_hbm.at[idx_v.at[pl.ds(off, _CHUNK)]], bufs[b], gsem
            ).wait()
            pltpu.async_copy(
                bufs[b], out_hbm.at[pl.ds(base + off, _CHUNK)], osems[b]
            )

        def body(g2, carry):
            for b in range(2):
                g = g2 * 2 + b
                @pl.when(g2 > 0)
                def _():
                    pltpu.make_async_copy(
                        bufs[b], out_hbm.at[pl.ds(base, _CHUNK)], osems[b]
                    ).wait()
                chunk(g, b)
            return carry

        lax.fori_loop(0, n_chunks // 2, body, 0)
        for b in range(2):
            pltpu.make_async_copy(
                bufs[b], out_hbm.at[pl.ds(base, _CHUNK)], osems[b]
            ).wait()

    return gather


def kernel(input, weight):
    B, S = input.shape
    _, D = weight.shape
    positions = _compute_positions(input)
    flat_pos = positions.reshape(B * S)
    out = _make_gather(B * S, D)(flat_pos, weight)
    return out.reshape(B, S, D)


# speedup vs baseline: 1.7966x; 1.7966x over previous
import functools

import jax
import jax.numpy as jnp
from jax import lax
from jax.experimental import pallas as pl
from jax.experimental.pallas import tpu as pltpu
from jax.experimental.pallas import tpu_sc as plsc

_PAD = 1
_NC = 2
_NS = 16
_NW = _NC * _NS
_CHUNK = 16


def _positions_body(tok_ref, pos_ref):
    t = tok_ref[...]
    B, S = t.shape
    mask = (t != _PAD).astype(jnp.int32)
    cs = mask
    s = 1
    while s < S:
        shifted = jnp.concatenate(
            [jnp.zeros((B, s), jnp.int32), cs[:, :-s]], axis=1)
        cs = cs + shifted
        s *= 2
    pos_ref[...] = cs * mask + _PAD


def _compute_positions(tokens):
    B, S = tokens.shape
    return pl.pallas_call(
        _positions_body,
        out_shape=jax.ShapeDtypeStruct((B, S), jnp.int32),
    )(tokens)


@functools.lru_cache(maxsize=None)
def _make_gather(n_rows, dim):
    b_per_w = n_rows // _NW
    n_chunks = b_per_w // _CHUNK
    mesh = plsc.VectorSubcoreMesh(
        core_axis_name="c", subcore_axis_name="s",
        num_cores=_NC, num_subcores=_NS)

    @functools.partial(
        pl.kernel,
        out_type=jax.ShapeDtypeStruct((n_rows, dim), jnp.float32),
        mesh=mesh,
        scratch_types=[
            pltpu.VMEM((b_per_w,), jnp.int32),
            pltpu.VMEM((_CHUNK, dim), jnp.float32),
            pltpu.VMEM((_CHUNK, dim), jnp.float32),
            pltpu.SemaphoreType.DMA,
            pltpu.SemaphoreType.DMA,
            pltpu.SemaphoreType.DMA,
        ],
    )
    def gather(pos_hbm, table_hbm, out_hbm, idx_v, buf0, buf1, gsem, osem0, osem1):
        wid = lax.axis_index("s") * _NC + lax.axis_index("c")
        base = wid * b_per_w
        pltpu.sync_copy(pos_hbm.at[pl.ds(base, b_per_w)], idx_v)

        bufs = (buf0, buf1)
        osems = (osem0, osem1)

        def chunk(g, b):
            off = g * _CHUNK
            pltpu.async_copy(
                table_hbm.at[idx_v.at[pl.ds(off, _CHUNK)]], bufs[b], gsem
            ).wait()
            pltpu.async_copy(
                bufs[b], out_hbm.at[pl.ds(base + off, _CHUNK)], osems[b]
            )

        def body(g2, carry):
            for b in range(2):
                g = g2 * 2 + b
                @pl.when(g2 > 0)
                def _():
                    pltpu.make_async_copy(
                        bufs[b], out_hbm.at[pl.ds(base, _CHUNK)], osems[b]
                    ).wait()
                chunk(g, b)
            return carry

        lax.fori_loop(0, n_chunks // 2, body, 0)
        for b in range(2):
            pltpu.make_async_copy(
                bufs[b], out_hbm.at[pl.ds(base, _CHUNK)], osems[b]
            ).wait()

    return gather


def kernel(input, weight):
    B, S = input.shape
    _, D = weight.shape
    positions = _compute_positions(input)
    flat_pos = positions.reshape(B * S)
    out = _make_gather(B * S, D)(flat_pos, weight)
    return out.reshape(B, S, D)


# speedup vs baseline: 1.8445x; 1.0266x over previous
import functools

import jax
import jax.numpy as jnp
from jax import lax
from jax.experimental import pallas as pl
from jax.experimental.pallas import tpu as pltpu
from jax.experimental.pallas import tpu_sc as plsc

_PAD = 1
_NC = 2
_NS = 16
_NW = _NC * _NS
_CHUNK = 8
_NSLOT = 4


def _positions_body(tok_ref, pos_ref):
    t = tok_ref[...]
    B, S = t.shape
    mask = (t != _PAD).astype(jnp.int32)
    cs = mask
    s = 1
    while s < S:
        shifted = jnp.concatenate(
            [jnp.zeros((B, s), jnp.int32), cs[:, :-s]], axis=1)
        cs = cs + shifted
        s *= 2
    pos_ref[...] = cs * mask + _PAD


def _compute_positions(tokens):
    B, S = tokens.shape
    return pl.pallas_call(
        _positions_body,
        out_shape=jax.ShapeDtypeStruct((B, S), jnp.int32),
    )(tokens)


@functools.lru_cache(maxsize=None)
def _make_gather(n_rows, dim):
    b_per_w = n_rows // _NW
    n_chunks = b_per_w // _CHUNK
    mesh = plsc.VectorSubcoreMesh(
        core_axis_name="c", subcore_axis_name="s",
        num_cores=_NC, num_subcores=_NS)

    @functools.partial(
        pl.kernel,
        out_type=jax.ShapeDtypeStruct((n_rows, dim), jnp.float32),
        mesh=mesh,
        scratch_types=[
            pltpu.VMEM((b_per_w,), jnp.int32),
            [pltpu.VMEM((_CHUNK, dim), jnp.float32) for _ in range(_NSLOT)],
            [pltpu.SemaphoreType.DMA for _ in range(_NSLOT)],
            [pltpu.SemaphoreType.DMA for _ in range(_NSLOT)],
        ],
    )
    def gather(pos_hbm, table_hbm, out_hbm, idx_v, bufs, gsems, osems):
        wid = lax.axis_index("s") * _NC + lax.axis_index("c")
        base = wid * b_per_w
        pltpu.sync_copy(pos_hbm.at[pl.ds(base, b_per_w)], idx_v)

        def issue_gather(g, b):
            pltpu.async_copy(
                table_hbm.at[idx_v.at[pl.ds(g * _CHUNK, _CHUNK)]],
                bufs[b], gsems[b])

        def wait_gather(b):
            pltpu.make_async_copy(table_hbm.at[pl.ds(0, _CHUNK)],
                                  bufs[b], gsems[b]).wait()

        def issue_store(g, b):
            pltpu.async_copy(
                bufs[b], out_hbm.at[pl.ds(base + g * _CHUNK, _CHUNK)],
                osems[b])

        def wait_store(b):
            pltpu.make_async_copy(bufs[b], out_hbm.at[pl.ds(base, _CHUNK)],
                                  osems[b]).wait()

        issue_gather(0, 0)
        issue_gather(1, 1)

        def step(g, b):
            wait_gather(b)
            issue_store(g, b)

        for g in range(2):
            step(g, g % _NSLOT)
            issue_gather(g + 2, (g + 2) % _NSLOT)

        n_steady = n_chunks - 4
        assert n_steady % _NSLOT == 0 and n_chunks >= 8

        def body(i, carry):
            for u in range(_NSLOT):
                g = 2 + i * _NSLOT + u
                b = (2 + u) % _NSLOT
                b2 = (4 + u) % _NSLOT
                step(g, b)
                wait_store(b2)
                issue_gather(g + 2, b2)
            return carry

        lax.fori_loop(0, n_steady // _NSLOT, body, 0)

        for g in range(n_chunks - 2, n_chunks):
            step(g, g % _NSLOT)

        for b in range(_NSLOT):
            wait_store(b)

    return gather


def kernel(input, weight):
    B, S = input.shape
    V, D = weight.shape
    positions = _compute_positions(input)
    flat_pos = positions.reshape(B * S)
    out = _make_gather(B * S, D)(flat_pos, weight)
    return out.reshape(B, S, D)
